# baseline (speedup 1.0000x reference)
import dataclasses
import functools

import jax
import jax.numpy as jnp
from jax import lax
from jax.experimental import pallas as pl
from jax.experimental.pallas import tpu as pltpu
from jax.experimental.pallas import tpu_sc as plsc

K_TOP = 64
CHUNK = 128
SUB = 16
CAPC = 80
CAPS = 160
CAPE = 256
NLANE = 16


def _encode_body(a_ref, w_ref, be_ref, s_ref, s3_ref):
    acc = jax.lax.dot_general(
        a_ref[...].astype(jnp.bfloat16), w_ref[...].astype(jnp.bfloat16),
        (((1,), (1,)), ((), ())),
        preferred_element_type=jnp.float32,
    )
    s = jnp.maximum(acc + be_ref[...], 0.0)
    s_ref[...] = s
    br, bm = s.shape
    s3_ref[...] = s.reshape(br, bm // CHUNK, CHUNK)


def _encode(a, w_e, b_e, br, bm):
    n, c = a.shape
    m = w_e.shape[0]
    grid = (n // br, m // bm)
    return pl.pallas_call(
        _encode_body,
        grid=grid,
        in_specs=[
            pl.BlockSpec((br, c), lambda i, j: (i, 0)),
            pl.BlockSpec((bm, c), lambda i, j: (j, 0)),
            pl.BlockSpec((1, bm), lambda i, j: (0, j)),
        ],
        out_specs=[
            pl.BlockSpec((br, bm), lambda i, j: (i, j)),
            pl.BlockSpec((br, bm // CHUNK, CHUNK), lambda i, j: (i, j, 0)),
        ],
        out_shape=[
            jax.ShapeDtypeStruct((n, m), jnp.float32),
            jax.ShapeDtypeStruct((n, m // CHUNK, CHUNK), jnp.float32),
        ],
        compiler_params=pltpu.CompilerParams(
            dimension_semantics=("parallel", "parallel"),
        ),
    )(a, w_e, b_e.reshape(1, m))


def _thresh_body(s_ref, cm128_ref, t1_ref):
    br, m = s_ref.shape
    cm128 = jnp.max(s_ref[...].reshape(br, m // CHUNK, CHUNK), axis=2)
    cm128_ref[...] = cm128

    def body(_, carry):
        cur, _m = carry
        mx = jnp.max(cur, axis=1, keepdims=True)
        cur = jnp.where(cur == mx, jnp.float32(-1.0), cur)
        return cur, mx

    _, t1 = jax.lax.fori_loop(0, K_TOP, body,
                              (cm128, jnp.zeros((br, 1), jnp.float32)))
    t1_ref[...] = jnp.broadcast_to(t1, (br, CHUNK))


def _thresh(s, bt):
    n, m = s.shape
    return pl.pallas_call(
        _thresh_body,
        grid=(n // bt,),
        in_specs=[pl.BlockSpec((bt, m), lambda i: (i, 0))],
        out_specs=[
            pl.BlockSpec((bt, m // CHUNK), lambda i: (i, 0)),
            pl.BlockSpec((bt, CHUNK), lambda i: (i, 0)),
        ],
        out_shape=[
            jax.ShapeDtypeStruct((n, m // CHUNK), jnp.float32),
            jax.ShapeDtypeStruct((n, CHUNK), jnp.float32),
        ],
        compiler_params=pltpu.CompilerParams(
            dimension_semantics=("parallel",),
        ),
    )(s)


def _bmerge(xk, xv, keep, skv):
    n = len(xk)
    if n == 1:
        k, v = skv(xk[0], xv[0])
        return [k], [v]
    h = n // 2
    hik, hiv, lok, lov = [], [], [], []
    for i in range(h):
        a, b, av, bv = xk[i], xk[i + h], xv[i], xv[i + h]
        t = a >= b
        hik.append(jnp.where(t, a, b))
        hiv.append(jnp.where(t, av, bv))
        lok.append(jnp.where(t, b, a))
        lov.append(jnp.where(t, bv, av))
    if keep <= h:
        return _bmerge(hik, hiv, keep, skv)
    hk, hv = _bmerge(hik, hiv, h, skv)
    lk, lv = _bmerge(lok, lov, keep - h, skv)
    return hk + lk, hv + lv


def _merge_runs(ak, av, bk, bv, keep, skv):
    xk = ak + [lax.rev(k, (0,)) for k in bk[::-1]]
    xv = av + [lax.rev(v, (0,)) for v in bv[::-1]]
    return _bmerge(xk, xv, keep, skv)


def _topk_sortnet(ks, vs, keep, skv):
    runs = []
    for k, v in zip(ks, vs):
        k1, v1 = skv(k, v)
        runs.append(([k1], [v1]))
    while len(runs) > 1:
        nxt = []
        for i in range(0, len(runs), 2):
            (ak, av), (bk, bv) = runs[i], runs[i + 1]
            cap = min(keep, len(ak) + len(bk))
            nxt.append(_merge_runs(ak, av, bk, bv, cap, skv))
        runs = nxt
    return runs[0]


def _skv_sc(k, v):
    return plsc.sort_key_val(k, v, descending=True)


def _sc_compiler_params():
    cp = pltpu.CompilerParams()
    if "needs_layout_passes" in pltpu.CompilerParams.__dataclass_fields__:
        cp = dataclasses.replace(cp, needs_layout_passes=False)
    return cp


def _sc_topk(s3, cm128, t1b):
    n = s3.shape[0]
    m = s3.shape[1] * s3.shape[2]
    nch = m // CHUNK
    stab = s3.reshape(n * nch, CHUNK)
    cm128f = cm128.reshape(-1)
    t1f = t1b.reshape(-1)
    mesh = plsc.VectorSubcoreMesh(core_axis_name="c", subcore_axis_name="s",
                                  num_cores=2, num_subcores=16)
    nworkers = 32
    rows_per = n // nworkers
    ngroups = rows_per // 8

    @functools.partial(
        pl.kernel,
        out_type=[
            jax.ShapeDtypeStruct((n * CHUNK,), jnp.float32),
            jax.ShapeDtypeStruct((n * CHUNK,), jnp.int32),
        ],
        mesh=mesh,
        scratch_types=[
            pltpu.VMEM((8 * nch,), jnp.float32),
            pltpu.VMEM((8 * CHUNK,), jnp.float32),
            pltpu.VMEM((8 * CAPC,), jnp.int32),
            pltpu.VMEM((8 * NLANE,), jnp.int32),
            pltpu.VMEM((8 * CAPC, CHUNK), jnp.float32),
            pltpu.VMEM((CAPE,), jnp.float32),
            pltpu.VMEM((CAPE,), jnp.int32),
            pltpu.VMEM((8 * CHUNK,), jnp.float32),
            pltpu.VMEM((8 * CHUNK,), jnp.int32),
            pltpu.SemaphoreType.DMA,
        ],
        compiler_params=_sc_compiler_params(),
    )
    def k(s_hbm, cm128_hbm, t1_hbm, vals_hbm, idx_hbm,
          cm128_v, t1_v, idb_v, cnt_v, g_v,
          valb_v, posb_v, outv_v, outi_v, sem):
        wid = lax.axis_index("s") * 2 + lax.axis_index("c")
        base_row = wid * rows_per
        iota = lax.iota(jnp.int32, NLANE)
        zero16 = jnp.zeros((NLANE,), jnp.int32)

        @pl.loop(0, ngroups)
        def _(grp):
            r0 = base_row + grp * 8
            pltpu.sync_copy(cm128_hbm.at[pl.ds(r0 * nch, 8 * nch)], cm128_v)
            pltpu.sync_copy(t1_hbm.at[pl.ds(r0 * CHUNK, 8 * CHUNK)], t1_v)

            @pl.loop(0, 8)
            def _(ri):
                r = r0 + ri
                t1 = t1_v[pl.ds(ri * CHUNK, NLANE)]
                for j in range(CAPC // NLANE):
                    idb_v[pl.ds(ri * CAPC + j * NLANE, NLANE)] = jnp.minimum(
                        r * nch + j * NLANE + iota, n * nch - 1)
                ptr = zero16
                for j in range(nch // NLANE):
                    v = cm128_v[pl.ds(ri * nch + j * NLANE, NLANE)]
                    mk = v >= t1
                    rank = plsc.cumsum(mk.astype(jnp.int32)) - 1 + ptr
                    rankc = jnp.minimum(rank, CAPC - 1)
                    pid = r * nch + j * NLANE + iota
                    plsc.store_scatter(idb_v, [ri * CAPC + rankc], pid,
                                       mask=mk)
                    ptr = ptr + plsc.all_reduce_population_count(mk)
                cnt_v[pl.ds(ri * NLANE, NLANE)] = jnp.minimum(ptr, CAPC)
                pltpu.async_copy(
                    s_hbm.at[idb_v.at[pl.ds(ri * CAPC, CAPC)]],
                    g_v.at[pl.ds(ri * CAPC, CAPC)], sem)

            @pl.loop(0, 8)
            def _(ri):
                pltpu.make_async_copy(
                    s_hbm.at[pl.ds(0, CAPC)],
                    g_v.at[pl.ds(ri * CAPC, CAPC)], sem).wait()

            @pl.loop(0, 8)
            def _(ri):
                r = r0 + ri
                t1 = t1_v[pl.ds(ri * CHUNK, NLANE)]
                nchunks = jnp.max(cnt_v[pl.ds(ri * NLANE, NLANE)])

                for j in range(CAPE // NLANE):
                    valb_v[pl.ds(j * NLANE, NLANE)] = jnp.full(
                        (NLANE,), -1.0, jnp.float32)
                    posb_v[pl.ds(j * NLANE, NLANE)] = jnp.full(
                        (NLANE,), 2 ** 30, jnp.int32)

                def fbody(j, p3):
                    js = zero16 + j
                    cid = plsc.load_gather(idb_v, [ri * CAPC + js])
                    pbase = (cid - r * nch) * CHUNK
                    for e in range(CHUNK // NLANE):
                        v = plsc.load_gather(
                            g_v, [ri * CAPC + js, e * NLANE + iota])
                        mk = v >= t1
                        pos = pbase + e * NLANE + iota
                        rank = plsc.cumsum(mk.astype(jnp.int32)) - 1 + p3
                        rankc = jnp.minimum(rank, CAPE - 1)
                        plsc.store_scatter(valb_v, [rankc], v, mask=mk)
                        plsc.store_scatter(posb_v, [rankc], pos, mask=mk)
                        p3 = p3 + plsc.all_reduce_population_count(mk)
                    return p3

                lax.fori_loop(0, nchunks, fbody, zero16)

                ks = [valb_v[pl.ds(j * NLANE, NLANE)]
                      for j in range(CAPE // NLANE)]
                vs = [posb_v[pl.ds(j * NLANE, NLANE)]
                      for j in range(CAPE // NLANE)]
                tk, tv = _topk_sortnet(ks, vs, K_TOP // NLANE, _skv_sc)
                for j in range(K_TOP // NLANE):
                    outv_v[pl.ds(ri * CHUNK + j * NLANE, NLANE)] = tk[j]
                    outi_v[pl.ds(ri * CHUNK + j * NLANE, NLANE)] = tv[j]

            pltpu.sync_copy(outv_v, vals_hbm.at[pl.ds(r0 * CHUNK,
                                                      8 * CHUNK)])
            pltpu.sync_copy(outi_v, idx_hbm.at[pl.ds(r0 * CHUNK,
                                                     8 * CHUNK)])

    return k(stab, cm128f, t1f)


def _decode_body(s_ref, d_ref, t_ref, bd_ref, out_ref):
    kb = pl.program_id(1)
    cur = s_ref[...]
    z = (cur * (cur >= t_ref[...])).astype(jnp.bfloat16)
    part = jax.lax.dot_general(
        z, d_ref[...].astype(jnp.bfloat16),
        (((1,), (1,)), ((), ())),
        preferred_element_type=jnp.float32,
    )

    @pl.when(kb == 0)
    def _():
        out_ref[...] = part + bd_ref[...]

    @pl.when(kb > 0)
    def _():
        out_ref[...] += part


def _decode(s, d, t64, b_d, br, bk):
    n, m = s.shape
    c = d.shape[0]
    grid = (n // br, m // bk)
    return pl.pallas_call(
        _decode_body,
        grid=grid,
        in_specs=[
            pl.BlockSpec((br, bk), lambda i, j: (i, j)),
            pl.BlockSpec((c, bk), lambda i, j: (0, j)),
            pl.BlockSpec((br, 1), lambda i, j: (i, 0)),
            pl.BlockSpec((1, c), lambda i, j: (0, 0)),
        ],
        out_specs=pl.BlockSpec((br, c), lambda i, j: (i, 0)),
        out_shape=jax.ShapeDtypeStruct((n, c), jnp.float32),
        compiler_params=pltpu.CompilerParams(
            dimension_semantics=("parallel", "arbitrary"),
        ),
    )(s, d, t64, b_d.reshape(1, c))


def kernel(a, W_e, b_e, D, b_d):
    n, c = a.shape
    m = W_e.shape[0]

    s, s3 = _encode(a, W_e, b_e, min(1024, n), min(1024, m))
    cm128, t1b = _thresh(s, min(128, n))
    vals_p, idx_p = _sc_topk(s3, cm128, t1b)
    vals = vals_p.reshape(n, CHUNK)[:, :K_TOP]
    idx = idx_p.reshape(n, CHUNK)[:, :K_TOP]

    t64 = vals[:, K_TOP - 1:K_TOP]
    recon = _decode(s, D, t64, b_d, min(512, n), min(1024, m))
    return (recon, vals, idx)
